# baseline (speedup 1.0000x reference)
import functools

import jax
import jax.numpy as jnp
from jax import lax
from jax.experimental import pallas as pl
from jax.experimental.pallas import tpu as pltpu
from jax.experimental.pallas import tpu_sc as plsc

VOCAB = 100000
EMBED = 64
HID = 16
OUT = 2
BATCH = 16384
HIST = 200
CHUNK_A = 104
CHUNK_B = 96
PACK = 16


def _mm_body(t_ref, w_ref, o_ref):
    d = jnp.dot(t_ref[0], w_ref[:], preferred_element_type=jnp.float32)
    hb = d.astype(jnp.bfloat16)
    u = lax.bitcast_convert_type(hb, jnp.uint16)
    a32 = u[:, :128].astype(jnp.uint32)
    b32 = u[:, 128:].astype(jnp.uint32)
    w = a32 | (b32 << 16)
    o_ref[:] = lax.bitcast_convert_type(w, jnp.float32)[None]


def _fold_table(table, W1):
    grid = 25
    pk_rows = VOCAB // PACK // grid
    t16 = table.reshape(grid, pk_rows, PACK * EMBED)
    w1big = jnp.kron(jnp.eye(PACK, dtype=W1.dtype), W1)
    half = PACK * HID // 2
    lane = jnp.arange(PACK * HID)
    perm = jnp.where(lane < half, 2 * lane, 2 * (lane - half) + 1)
    w1bigp = w1big[:, perm]
    packed = pl.pallas_call(
        _mm_body,
        grid=(grid,),
        in_specs=[
            pl.BlockSpec((1, pk_rows, PACK * EMBED), lambda i: (i, 0, 0)),
            pl.BlockSpec((PACK * EMBED, PACK * HID), lambda i: (0, 0)),
        ],
        out_specs=pl.BlockSpec((1, pk_rows, PACK * HID // 2), lambda i: (i, 0, 0)),
        out_shape=jax.ShapeDtypeStruct((grid, pk_rows, PACK * HID // 2), jnp.float32),
    )(t16, w1bigp)
    t1b = lax.bitcast_convert_type(packed, jnp.bfloat16)
    return t1b.reshape(VOCAB, HID)


NBUF = 4


def _make_sc_pool():
    info = plsc.get_sparse_core_info()
    nc, ns = info.num_cores, info.num_subcores
    nw = nc * ns
    bpw = BATCH // nw
    mesh = plsc.VectorSubcoreMesh(core_axis_name="c", subcore_axis_name="s")

    @functools.partial(
        pl.kernel,
        out_type=jax.ShapeDtypeStruct((BATCH, 2, HID), jnp.bfloat16),
        mesh=mesh,
        scratch_types=[
            pltpu.VMEM((bpw, HIST), jnp.int32),
            pltpu.VMEM((NBUF, HIST, HID), jnp.bfloat16),
            pltpu.VMEM((bpw, 2, HID), jnp.bfloat16),
            [pltpu.SemaphoreType.DMA] * NBUF,
        ],
        compiler_params=pltpu.CompilerParams(
            use_tc_tiling_on_sc=False, needs_layout_passes=False
        ),
    )
    def sc_pool(x_hbm, t1_hbm, out_hbm, idx_v, bufs, out_v, sems):
        wid = lax.axis_index("s") * nc + lax.axis_index("c")
        base = wid * bpw
        pltpu.sync_copy(x_hbm.at[pl.ds(base, bpw)], idx_v)

        def issue(slot, r):
            pltpu.async_copy(
                t1_hbm.at[idx_v.at[r, pl.ds(0, CHUNK_A)]],
                bufs.at[slot, pl.ds(0, CHUNK_A)],
                sems[slot],
            )
            pltpu.async_copy(
                t1_hbm.at[idx_v.at[r, pl.ds(CHUNK_A, CHUNK_B)]],
                bufs.at[slot, pl.ds(CHUNK_A, CHUNK_B)],
                sems[slot],
            )

        def drain(slot):
            pltpu.make_async_copy(
                t1_hbm.at[idx_v.at[0, pl.ds(0, CHUNK_A)]],
                bufs.at[slot, pl.ds(0, CHUNK_A)],
                sems[slot],
            ).wait()
            pltpu.make_async_copy(
                t1_hbm.at[idx_v.at[0, pl.ds(CHUNK_A, CHUNK_B)]],
                bufs.at[slot, pl.ds(CHUNK_A, CHUNK_B)],
                sems[slot],
            ).wait()

        for b in range(NBUF):
            issue(b, b)

        def outer(r0, _):
            for b in range(NBUF):
                r = r0 + b
                drain(b)
                stack = []
                for g in range(HIST // 2):
                    v = bufs[b, pl.ds(2 * g, 2), :]
                    k = g
                    while k % 2 == 1:
                        v = stack.pop() + v
                        k //= 2
                    stack.append(v)
                acc = stack[0]
                for v in stack[1:]:
                    acc = acc + v
                out_v[r] = acc

                @pl.when(r + NBUF < bpw)
                def _():
                    issue(b, r + NBUF)

            return 0

        lax.fori_loop(0, bpw // NBUF, lambda i, c: outer(i * NBUF, c), 0)
        pltpu.sync_copy(out_v, out_hbm.at[pl.ds(base, bpw)])

    return sc_pool


def _head_body(s_ref, b1_ref, w2_ref, b2_ref, o_ref):
    pooled = s_ref[:, 0, :].astype(jnp.float32) + s_ref[:, 1, :].astype(jnp.float32)
    h = jnp.maximum(pooled * (1.0 / HIST) + b1_ref[:], 0.0)
    o_ref[:] = jnp.dot(h, w2_ref[:], preferred_element_type=jnp.float32) + b2_ref[:]


def _head(sums, b1, W2, b2):
    rows_blk = 2048
    grid = BATCH // rows_blk
    return pl.pallas_call(
        _head_body,
        grid=(grid,),
        in_specs=[
            pl.BlockSpec((rows_blk, 2, HID), lambda i: (i, 0, 0)),
            pl.BlockSpec((1, HID), lambda i: (0, 0)),
            pl.BlockSpec((HID, OUT), lambda i: (0, 0)),
            pl.BlockSpec((1, OUT), lambda i: (0, 0)),
        ],
        out_specs=pl.BlockSpec((rows_blk, OUT), lambda i: (i, 0)),
        out_shape=jax.ShapeDtypeStruct((BATCH, OUT), jnp.float32),
    )(sums, b1.reshape(1, HID), W2, b2.reshape(1, OUT))


def kernel(x, table, W1, b1, W2, b2):
    table1 = _fold_table(table, W1)
    sums = _make_sc_pool()(x.astype(jnp.int32), table1)
    return _head(sums, b1, W2, b2)


# speedup vs baseline: 2.2671x; 2.2671x over previous
import functools

import jax
import jax.numpy as jnp
from jax import lax
from jax.experimental import pallas as pl
from jax.experimental.pallas import tpu as pltpu
from jax.experimental.pallas import tpu_sc as plsc

VOCAB = 100000
EMBED = 64
HID = 16
OUT = 2
BATCH = 16384
HIST = 200
CHUNK_A = 104
CHUNK_B = 96
PACK = 128 // HID


def _mm_body(t_ref, w_ref, o_ref):
    o_ref[:] = jnp.dot(t_ref[0], w_ref[:], preferred_element_type=jnp.float32)[None]


def _fold_table(table, W1):
    grid = 50
    pk_rows = VOCAB // PACK // grid
    t8 = table.reshape(grid, pk_rows, PACK * EMBED)
    w1big = jnp.kron(jnp.eye(PACK, dtype=W1.dtype), W1)
    packed = pl.pallas_call(
        _mm_body,
        grid=(grid,),
        in_specs=[
            pl.BlockSpec((1, pk_rows, PACK * EMBED), lambda i: (i, 0, 0)),
            pl.BlockSpec((PACK * EMBED, PACK * HID), lambda i: (0, 0)),
        ],
        out_specs=pl.BlockSpec((1, pk_rows, PACK * HID), lambda i: (i, 0, 0)),
        out_shape=jax.ShapeDtypeStruct((grid, pk_rows, PACK * HID), jnp.float32),
    )(t8, w1big)
    return packed.reshape(VOCAB, HID)


NBUF = 4


def _make_sc_pool():
    info = plsc.get_sparse_core_info()
    nc, ns = info.num_cores, info.num_subcores
    nw = nc * ns
    bpw = BATCH // nw
    mesh = plsc.VectorSubcoreMesh(core_axis_name="c", subcore_axis_name="s")

    @functools.partial(
        pl.kernel,
        out_type=jax.ShapeDtypeStruct((BATCH, HID), jnp.float32),
        mesh=mesh,
        scratch_types=[
            pltpu.VMEM((bpw * HIST,), jnp.int32),
            pltpu.VMEM((NBUF, HIST, HID), jnp.float32),
            pltpu.VMEM((bpw, HID), jnp.float32),
            [pltpu.SemaphoreType.DMA] * NBUF,
        ],
        compiler_params=pltpu.CompilerParams(use_tc_tiling_on_sc=False),
    )
    def sc_pool(x_hbm, t1_hbm, out_hbm, idx_v, bufs, out_v, sems):
        wid = lax.axis_index("s") * nc + lax.axis_index("c")
        base = wid * bpw
        pltpu.sync_copy(x_hbm.at[pl.ds(base * HIST, bpw * HIST)], idx_v)

        def issue(slot, r):
            pltpu.async_copy(
                t1_hbm.at[idx_v.at[pl.ds(r * HIST, CHUNK_A)]],
                bufs.at[slot, pl.ds(0, CHUNK_A)],
                sems[slot],
            )
            pltpu.async_copy(
                t1_hbm.at[idx_v.at[pl.ds(r * HIST + CHUNK_A, CHUNK_B)]],
                bufs.at[slot, pl.ds(CHUNK_A, CHUNK_B)],
                sems[slot],
            )

        def drain(slot):
            pltpu.make_async_copy(
                t1_hbm.at[idx_v.at[pl.ds(0, CHUNK_A)]],
                bufs.at[slot, pl.ds(0, CHUNK_A)],
                sems[slot],
            ).wait()
            pltpu.make_async_copy(
                t1_hbm.at[idx_v.at[pl.ds(0, CHUNK_B)]],
                bufs.at[slot, pl.ds(CHUNK_A, CHUNK_B)],
                sems[slot],
            ).wait()

        for b in range(NBUF):
            issue(b, b)

        def outer(r0, _):
            for b in range(NBUF):
                r = r0 + b
                drain(b)
                accs = [jnp.zeros((HID,), jnp.float32)] * 8
                for j in range(HIST):
                    accs[j % 8] = accs[j % 8] + bufs[b, j]
                out_v[r] = (
                    ((accs[0] + accs[1]) + (accs[2] + accs[3]))
                    + ((accs[4] + accs[5]) + (accs[6] + accs[7]))
                )

                @pl.when(r + NBUF < bpw)
                def _():
                    issue(b, r + NBUF)

            return 0

        lax.fori_loop(0, bpw // NBUF, lambda i, c: outer(i * NBUF, c), 0)
        pltpu.sync_copy(out_v, out_hbm.at[pl.ds(base, bpw)])

    return sc_pool


def _head_body(s_ref, b1_ref, w2_ref, b2_ref, o_ref):
    h = jnp.maximum(s_ref[:] * (1.0 / HIST) + b1_ref[:], 0.0)
    o_ref[:] = jnp.dot(h, w2_ref[:], preferred_element_type=jnp.float32) + b2_ref[:]


def _head(sums, b1, W2, b2):
    rows_blk = 2048
    grid = BATCH // rows_blk
    return pl.pallas_call(
        _head_body,
        grid=(grid,),
        in_specs=[
            pl.BlockSpec((rows_blk, HID), lambda i: (i, 0)),
            pl.BlockSpec((1, HID), lambda i: (0, 0)),
            pl.BlockSpec((HID, OUT), lambda i: (0, 0)),
            pl.BlockSpec((1, OUT), lambda i: (0, 0)),
        ],
        out_specs=pl.BlockSpec((rows_blk, OUT), lambda i: (i, 0)),
        out_shape=jax.ShapeDtypeStruct((BATCH, OUT), jnp.float32),
    )(sums, b1.reshape(1, HID), W2, b2.reshape(1, OUT))


def kernel(x, table, W1, b1, W2, b2):
    table1 = _fold_table(table, W1)
    sums = _make_sc_pool()(x.astype(jnp.int32).reshape(-1), table1)
    return _head(sums, b1, W2, b2)
